# Initial kernel scaffold; baseline (speedup 1.0000x reference)
import jax
import jax.numpy as jnp
from jax.experimental import pallas as pl


def kernel(belief, state, Wb, Ws, Wa, Wz, W1, w2):
    raise NotImplementedError("write your pallas kernel here")


# speedup vs baseline: 1.6744x; 1.6744x over previous
import jax
import jax.numpy as jnp
from jax.experimental import pallas as pl

B = 16
H = 512
Z = 128
A = 8
CAND = 256
ITERS = 2
T = 12
TOPK = 32
MAXA = 1.0
MINA = -1.0
D = 512


def _rollout_body(noise_ref, mean_ref, std_ref, belief_ref, state_ref,
                  wfull_ref, wz_ref, w2_ref, ret_ref, act_ref):
    b = jnp.broadcast_to(belief_ref[0], (CAND, H))
    s = jnp.broadcast_to(state_ref[0], (CAND, Z))
    mean = mean_ref[0]
    std = std_ref[0]
    hacc = jnp.zeros((CAND, D), jnp.float32)
    for t in range(T):
        a_t = jnp.clip(mean[t][None, :] + std[t][None, :] * noise_ref[t],
                       MINA, MAXA)
        act_ref[t] = a_t
        x = jnp.concatenate([b, s, a_t], axis=1)
        if t == 0:
            pre_b = jnp.dot(x, wfull_ref[:, :H],
                            preferred_element_type=jnp.float32)
        else:
            y = jnp.dot(x, wfull_ref[:], preferred_element_type=jnp.float32)
            pre_b = y[:, :H]
            hacc = hacc + jnp.tanh(y[:, H:])
        b = jnp.tanh(pre_b)
        s = jnp.tanh(jnp.dot(b, wz_ref[:], preferred_element_type=jnp.float32))
    xf = jnp.concatenate([b, s], axis=1)
    hacc = hacc + jnp.tanh(jnp.dot(xf, wfull_ref[:H + Z, H:],
                                   preferred_element_type=jnp.float32))
    ret_ref[0, 0, :] = jnp.sum(hacc * w2_ref[:], axis=1)


def _moments_body(ret_ref, act_ref, mean_ref, std_ref):
    r = ret_ref[:, 0, :]
    iota = jax.lax.broadcasted_iota(jnp.int32, (B, CAND), 1)
    w = jnp.zeros((B, CAND), jnp.float32)
    for _ in range(TOPK):
        m = jnp.max(r, axis=1, keepdims=True)
        is_max = r == m
        idx = jnp.min(jnp.where(is_max, iota, CAND), axis=1, keepdims=True)
        first = iota == idx
        w = jnp.where(first, 1.0, w)
        r = jnp.where(first, -jnp.inf, r)
    w3 = w[:, :, None]
    inv_k = 1.0 / TOPK
    for t in range(T):
        at = act_ref[t].reshape(B, CAND, A)
        mean_t = jnp.sum(at * w3, axis=1) * inv_k
        sq_t = jnp.sum(at * at * w3, axis=1) * inv_k
        mean_ref[:, t, :] = mean_t
        std_ref[:, t, :] = jnp.sqrt(jnp.maximum(sq_t - mean_t * mean_t, 0.0))


def _rollout(noise, mean, std, belief3, state3, wfull, wz, w2row):
    return pl.pallas_call(
        _rollout_body,
        grid=(B,),
        in_specs=[
            pl.BlockSpec((T, CAND, A), lambda i: (0, i, 0)),
            pl.BlockSpec((1, T, A), lambda i: (i, 0, 0)),
            pl.BlockSpec((1, T, A), lambda i: (i, 0, 0)),
            pl.BlockSpec((1, 1, H), lambda i: (i, 0, 0)),
            pl.BlockSpec((1, 1, Z), lambda i: (i, 0, 0)),
            pl.BlockSpec((H + Z + A, 2 * D), lambda i: (0, 0)),
            pl.BlockSpec((H, Z), lambda i: (0, 0)),
            pl.BlockSpec((1, D), lambda i: (0, 0)),
        ],
        out_specs=[
            pl.BlockSpec((1, 1, CAND), lambda i: (i, 0, 0)),
            pl.BlockSpec((T, CAND, A), lambda i: (0, i, 0)),
        ],
        out_shape=[
            jax.ShapeDtypeStruct((B, 1, CAND), jnp.float32),
            jax.ShapeDtypeStruct((T, B * CAND, A), jnp.float32),
        ],
    )(noise, mean, std, belief3, state3, wfull, wz, w2row)


def _moments(returns, actions):
    return pl.pallas_call(
        _moments_body,
        out_shape=[
            jax.ShapeDtypeStruct((B, T, A), jnp.float32),
            jax.ShapeDtypeStruct((B, T, A), jnp.float32),
        ],
    )(returns, actions)


def kernel(belief, state, Wb, Ws, Wa, Wz, W1, w2):
    key = jax.random.key(42)
    noises = []
    for _ in range(ITERS):
        key, sub = jax.random.split(key)
        noises.append(jax.random.normal(sub, (T, B, CAND, A),
                                        dtype=jnp.float32).reshape(T, B * CAND, A))
    wfull = jnp.concatenate([
        jnp.concatenate([Wb, Ws, Wa], axis=0),
        jnp.concatenate([W1, jnp.zeros((A, D), jnp.float32)], axis=0),
    ], axis=1)
    belief3 = belief.reshape(B, 1, H)
    state3 = state.reshape(B, 1, Z)
    w2row = w2.reshape(1, D)
    mean = jnp.zeros((B, T, A), jnp.float32)
    std = jnp.ones((B, T, A), jnp.float32)
    for i in range(ITERS):
        returns, actions = _rollout(noises[i], mean, std, belief3, state3,
                                    wfull, Wz, w2row)
        mean, std = _moments(returns, actions)
    return mean[:, 0, :]
